# baseline (speedup 1.0000x reference)
import functools

import jax
import jax.numpy as jnp
from jax import lax
from jax.experimental import pallas as pl
from jax.experimental.pallas import tpu as pltpu
from jax.experimental.pallas import tpu_sc as plsc

BATCH = 1024
VOCAB = 100000
EMBED = 32


_VT = 1024
_NT = pl.cdiv(VOCAB, _VT)


def _logits_body(w_ref, c_ref, b_ref, out_ref):
    out_ref[...] = lax.dot_general(
        w_ref[...],
        c_ref[...],
        dimension_numbers=(((0,), (0,)), ((), ())),
        preferred_element_type=jnp.float32,
    ) + b_ref[...]


def _logits_t(wt, ctx_t, bias_col):
    return pl.pallas_call(
        _logits_body,
        grid=(_NT,),
        in_specs=[
            pl.BlockSpec((EMBED, _VT), lambda j: (0, j)),
            pl.BlockSpec((EMBED, BATCH), lambda j: (0, 0)),
            pl.BlockSpec((_VT, 1), lambda j: (j, 0)),
        ],
        out_specs=pl.BlockSpec((_VT, BATCH), lambda j: (j, 0)),
        out_shape=jax.ShapeDtypeStruct((VOCAB, BATCH), jnp.float32),
        compiler_params=pltpu.CompilerParams(
            dimension_semantics=("parallel",),
        ),
    )(wt, ctx_t, bias_col)


try:
    _info = plsc.get_sparse_core_info()
    _NC, _NS = _info.num_cores, _info.num_subcores
except Exception:
    _NC, _NS = 2, 16
_NW = _NC * _NS
_BPW = BATCH // _NW

_sc_mesh = plsc.VectorSubcoreMesh(core_axis_name="c", subcore_axis_name="s")


@functools.partial(
    pl.kernel,
    mesh=_sc_mesh,
    out_type=jax.ShapeDtypeStruct((BATCH, EMBED), jnp.float32),
    scratch_types=[
        pltpu.VMEM((_BPW,), jnp.int32),
        pltpu.VMEM((_BPW, EMBED), jnp.float32),
        pltpu.SemaphoreType.DMA,
    ],
    compiler_params=pltpu.CompilerParams(use_tc_tiling_on_sc=False),
)
def _sc_gather(table_hbm, idx_hbm, out_hbm, idx_v, rows_v, sem):
    wid = lax.axis_index("s") * _NC + lax.axis_index("c")
    base = wid * _BPW
    pltpu.sync_copy(idx_hbm.at[pl.ds(base, _BPW)], idx_v)
    pltpu.async_copy(table_hbm.at[idx_v], rows_v, sem).wait()
    pltpu.sync_copy(rows_v, out_hbm.at[pl.ds(base, _BPW)])


def kernel(context_features, labels, label_embed_weight, out_fc_weight, out_fc_bias):
    logits_t = _logits_t(
        out_fc_weight.T,
        context_features.T,
        out_fc_bias.reshape(VOCAB, 1),
    )
    out_logits = logits_t.T
    out_embeddings = _sc_gather(label_embed_weight, labels.astype(jnp.int32))
    return (context_features, out_logits, out_embeddings, label_embed_weight)


# speedup vs baseline: 1.0947x; 1.0947x over previous
import functools

import jax
import jax.numpy as jnp
from jax import lax
from jax.experimental import pallas as pl
from jax.experimental.pallas import tpu as pltpu
from jax.experimental.pallas import tpu_sc as plsc

BATCH = 1024
VOCAB = 100000
EMBED = 32


_VT = 6144
_NT = pl.cdiv(VOCAB, _VT)


def _logits_body(w_ref, c_ref, b_ref, out_ref):
    out_ref[...] = lax.dot_general(
        w_ref[...],
        c_ref[...],
        dimension_numbers=(((0,), (0,)), ((), ())),
        preferred_element_type=jnp.float32,
    ) + b_ref[...]


def _logits_t(wt, ctx_t, bias_col):
    return pl.pallas_call(
        _logits_body,
        grid=(_NT,),
        in_specs=[
            pl.BlockSpec((EMBED, _VT), lambda j: (0, j)),
            pl.BlockSpec((EMBED, BATCH), lambda j: (0, 0)),
            pl.BlockSpec((_VT, 1), lambda j: (j, 0)),
        ],
        out_specs=pl.BlockSpec((_VT, BATCH), lambda j: (j, 0)),
        out_shape=jax.ShapeDtypeStruct((VOCAB, BATCH), jnp.float32),
        compiler_params=pltpu.CompilerParams(
            dimension_semantics=("parallel",),
            vmem_limit_bytes=110 * 1024 * 1024,
        ),
    )(wt, ctx_t, bias_col)


try:
    _info = plsc.get_sparse_core_info()
    _NC, _NS = _info.num_cores, _info.num_subcores
except Exception:
    _NC, _NS = 2, 16
_NW = _NC * _NS
_BPW = BATCH // _NW

_sc_mesh = plsc.VectorSubcoreMesh(core_axis_name="c", subcore_axis_name="s")


@functools.partial(
    pl.kernel,
    mesh=_sc_mesh,
    out_type=jax.ShapeDtypeStruct((BATCH, EMBED), jnp.float32),
    scratch_types=[
        pltpu.VMEM((_BPW,), jnp.int32),
        pltpu.VMEM((_BPW, EMBED), jnp.float32),
        pltpu.SemaphoreType.DMA,
    ],
    compiler_params=pltpu.CompilerParams(use_tc_tiling_on_sc=False),
)
def _sc_gather(table_hbm, idx_hbm, out_hbm, idx_v, rows_v, sem):
    wid = lax.axis_index("s") * _NC + lax.axis_index("c")
    base = wid * _BPW
    pltpu.sync_copy(idx_hbm.at[pl.ds(base, _BPW)], idx_v)
    pltpu.async_copy(table_hbm.at[idx_v], rows_v, sem).wait()
    pltpu.sync_copy(rows_v, out_hbm.at[pl.ds(base, _BPW)])


def kernel(context_features, labels, label_embed_weight, out_fc_weight, out_fc_bias):
    logits_t = _logits_t(
        out_fc_weight.T,
        context_features.T,
        out_fc_bias.reshape(VOCAB, 1),
    )
    out_logits = logits_t.T
    out_embeddings = _sc_gather(label_embed_weight, labels.astype(jnp.int32))
    return (context_features, out_logits, out_embeddings, label_embed_weight)


# speedup vs baseline: 1.3894x; 1.2692x over previous
import functools

import jax
import jax.numpy as jnp
from jax import lax
from jax.experimental import pallas as pl
from jax.experimental.pallas import tpu as pltpu
from jax.experimental.pallas import tpu_sc as plsc

BATCH = 1024
VOCAB = 100000
EMBED = 32


_VT = 4096
_NT = pl.cdiv(VOCAB, _VT)


def _logits_body(w_ref, c_ref, b_ref, out_ref):
    out_ref[...] = lax.dot_general(
        w_ref[...],
        c_ref[...],
        dimension_numbers=(((0,), (0,)), ((), ())),
        preferred_element_type=jnp.float32,
    ) + b_ref[...].T


def _logits_t(wt, ctx_t, bias_col):
    return pl.pallas_call(
        _logits_body,
        grid=(_NT,),
        in_specs=[
            pl.BlockSpec((EMBED, _VT), lambda j: (0, j)),
            pl.BlockSpec((EMBED, BATCH), lambda j: (0, 0)),
            pl.BlockSpec((1, _VT), lambda j: (0, j)),
        ],
        out_specs=pl.BlockSpec((_VT, BATCH), lambda j: (j, 0)),
        out_shape=jax.ShapeDtypeStruct((VOCAB, BATCH), jnp.float32),
        compiler_params=pltpu.CompilerParams(
            dimension_semantics=("parallel",),
            vmem_limit_bytes=110 * 1024 * 1024,
        ),
    )(wt, ctx_t, bias_col)


try:
    _info = plsc.get_sparse_core_info()
    _NC, _NS = _info.num_cores, _info.num_subcores
except Exception:
    _NC, _NS = 2, 16
_NW = _NC * _NS
_BPW = BATCH // _NW

_sc_mesh = plsc.VectorSubcoreMesh(core_axis_name="c", subcore_axis_name="s")


@functools.partial(
    pl.kernel,
    mesh=_sc_mesh,
    out_type=jax.ShapeDtypeStruct((BATCH, EMBED), jnp.float32),
    scratch_types=[
        pltpu.VMEM((_BPW,), jnp.int32),
        pltpu.VMEM((_BPW, EMBED), jnp.float32),
        pltpu.SemaphoreType.DMA,
    ],
    compiler_params=pltpu.CompilerParams(use_tc_tiling_on_sc=False),
)
def _sc_gather(table_hbm, idx_hbm, out_hbm, idx_v, rows_v, sem):
    wid = lax.axis_index("s") * _NC + lax.axis_index("c")
    base = wid * _BPW
    pltpu.sync_copy(idx_hbm.at[pl.ds(base, _BPW)], idx_v)
    pltpu.async_copy(table_hbm.at[idx_v], rows_v, sem).wait()
    pltpu.sync_copy(rows_v, out_hbm.at[pl.ds(base, _BPW)])


def kernel(context_features, labels, label_embed_weight, out_fc_weight, out_fc_bias):
    logits_t = _logits_t(
        out_fc_weight.T,
        context_features.T,
        out_fc_bias.reshape(1, VOCAB),
    )
    out_logits = logits_t.T
    out_embeddings = _sc_gather(label_embed_weight, labels.astype(jnp.int32))
    return (context_features, out_logits, out_embeddings, label_embed_weight)
